# Initial kernel scaffold; baseline (speedup 1.0000x reference)
import jax
import jax.numpy as jnp
from jax.experimental import pallas as pl


def kernel(x, embed_table):
    raise NotImplementedError("write your pallas kernel here")


# speedup vs baseline: 1.0234x; 1.0234x over previous
import functools

import jax
import jax.numpy as jnp
from jax import lax
from jax.experimental import pallas as pl
from jax.experimental.pallas import tpu as pltpu
from jax.experimental.pallas import tpu_sc as plsc

NUM_EMB = 1000000
D = 32
NC, NS = 2, 16
NW = NC * NS
B = 16384 * 50
BPW = B // NW
CHUNK = 128
NCHUNK = BPW // CHUNK

_mesh = plsc.VectorSubcoreMesh(
    core_axis_name="c", subcore_axis_name="s", num_cores=NC, num_subcores=NS
)


@functools.partial(
    pl.kernel,
    out_type=jax.ShapeDtypeStruct((B, D), jnp.float32),
    mesh=_mesh,
    scratch_types=[
        pltpu.VMEM((NCHUNK, CHUNK), jnp.int32),
        pltpu.VMEM((CHUNK, D), jnp.float32),
        pltpu.SemaphoreType.DMA,
    ],
    compiler_params=pltpu.CompilerParams(use_tc_tiling_on_sc=False),
)
def _gather(idx_hbm, table_hbm, out_hbm, idx_v, rows_v, sem):
    wid = lax.axis_index("s") * NC + lax.axis_index("c")
    pltpu.sync_copy(idx_hbm.at[wid], idx_v)
    base = wid * BPW

    @pl.loop(0, NCHUNK)
    def _step(j):
        pltpu.async_copy(table_hbm.at[idx_v.at[j]], rows_v, sem).wait()
        pltpu.sync_copy(rows_v, out_hbm.at[pl.ds(base + j * CHUNK, CHUNK)])


def kernel(x, embed_table):
    idx = x.astype(jnp.int32).reshape(NW, NCHUNK, CHUNK)
    out = _gather(idx, embed_table)
    return out.reshape(x.shape + (D,))


# speedup vs baseline: 1.1028x; 1.0776x over previous
import functools

import jax
import jax.numpy as jnp
from jax import lax
from jax.experimental import pallas as pl
from jax.experimental.pallas import tpu as pltpu
from jax.experimental.pallas import tpu_sc as plsc

NUM_EMB = 1000000
D = 32
NC, NS = 2, 16
NW = NC * NS
B = 16384 * 50
BPW = B // NW
CHUNK = 1024
NCHUNK = BPW // CHUNK

_mesh = plsc.VectorSubcoreMesh(
    core_axis_name="c", subcore_axis_name="s", num_cores=NC, num_subcores=NS
)


@functools.partial(
    pl.kernel,
    out_type=jax.ShapeDtypeStruct((B, D), jnp.float32),
    mesh=_mesh,
    scratch_types=[
        pltpu.VMEM((NCHUNK, CHUNK), jnp.int32),
        pltpu.VMEM((CHUNK, D), jnp.float32),
        pltpu.SemaphoreType.DMA,
    ],
    compiler_params=pltpu.CompilerParams(use_tc_tiling_on_sc=False),
)
def _gather(idx_hbm, table_hbm, out_hbm, idx_v, rows_v, sem):
    wid = lax.axis_index("s") * NC + lax.axis_index("c")
    pltpu.sync_copy(idx_hbm.at[wid], idx_v)
    base = wid * BPW

    @pl.loop(0, NCHUNK)
    def _step(j):
        pltpu.async_copy(table_hbm.at[idx_v.at[j]], rows_v, sem).wait()
        pltpu.sync_copy(rows_v, out_hbm.at[pl.ds(base + j * CHUNK, CHUNK)])


def kernel(x, embed_table):
    idx = x.astype(jnp.int32).reshape(NW, NCHUNK, CHUNK)
    out = _gather(idx, embed_table)
    return out.reshape(x.shape + (D,))


# speedup vs baseline: 1.1137x; 1.0099x over previous
import functools

import jax
import jax.numpy as jnp
from jax import lax
from jax.experimental import pallas as pl
from jax.experimental.pallas import tpu as pltpu
from jax.experimental.pallas import tpu_sc as plsc

NUM_EMB = 1000000
D = 32
NC, NS = 2, 16
NW = NC * NS
B = 16384 * 50
BPW = B // NW
CHUNK = 512
NCHUNK = BPW // CHUNK
NBUF = 5

_mesh = plsc.VectorSubcoreMesh(
    core_axis_name="c", subcore_axis_name="s", num_cores=NC, num_subcores=NS
)


@functools.partial(
    pl.kernel,
    out_type=jax.ShapeDtypeStruct((B, D), jnp.float32),
    mesh=_mesh,
    scratch_types=[
        pltpu.VMEM((NCHUNK, CHUNK), jnp.int32),
        pltpu.VMEM((NBUF, CHUNK, D), jnp.float32),
        pltpu.SemaphoreType.DMA((NBUF,)),
        pltpu.SemaphoreType.DMA((NBUF,)),
    ],
    compiler_params=pltpu.CompilerParams(use_tc_tiling_on_sc=False),
)
def _gather(idx_hbm, table_hbm, out_hbm, idx_v, rows_v, gsem, osem):
    wid = lax.axis_index("s") * NC + lax.axis_index("c")
    pltpu.sync_copy(idx_hbm.at[wid], idx_v)
    base = wid * BPW

    def gather_start(jj, b):
        pltpu.async_copy(table_hbm.at[idx_v.at[jj]], rows_v.at[b], gsem.at[b])

    def gather_wait(jj, b):
        pltpu.make_async_copy(
            table_hbm.at[idx_v.at[jj]], rows_v.at[b], gsem.at[b]
        ).wait()

    def out_start(jj, b):
        pltpu.async_copy(
            rows_v.at[b], out_hbm.at[pl.ds(base + jj * CHUNK, CHUNK)], osem.at[b]
        )

    def out_wait(jj, b):
        pltpu.make_async_copy(
            rows_v.at[b], out_hbm.at[pl.ds(base + jj * CHUNK, CHUNK)], osem.at[b]
        ).wait()

    for b in range(NBUF - 1):
        gather_start(b, b)

    @pl.loop(0, NCHUNK, step=NBUF)
    def _group(j):
        for b in range(NBUF):
            jj = j + b
            bn = (b + NBUF - 1) % NBUF
            jn = jj + NBUF - 1

            @pl.when(jn < NCHUNK)
            def _():
                @pl.when(jn >= NBUF)
                def _():
                    out_wait(jn - NBUF, bn)
                gather_start(jn, bn)

            gather_wait(jj, b)
            out_start(jj, b)

    for b in range(NBUF):
        out_wait(NCHUNK - NBUF + b, (NCHUNK - NBUF + b) % NBUF)


def kernel(x, embed_table):
    idx = x.astype(jnp.int32).reshape(NW, NCHUNK, CHUNK)
    out = _gather(idx, embed_table)
    return out.reshape(x.shape + (D,))


# speedup vs baseline: 1.9452x; 1.7466x over previous
import functools

import jax
import jax.numpy as jnp
from jax import lax
from jax.experimental import pallas as pl
from jax.experimental.pallas import tpu as pltpu
from jax.experimental.pallas import tpu_sc as plsc

NUM_EMB = 1000000
D = 32
NC, NS = 2, 16
NW = NC * NS
B = 16384 * 50
BPW = B // NW
CHUNK = 512
NCHUNK = BPW // CHUNK
NBUF = 5

_mesh = plsc.VectorSubcoreMesh(
    core_axis_name="c", subcore_axis_name="s", num_cores=NC, num_subcores=NS
)


@functools.partial(
    pl.kernel,
    out_type=jax.ShapeDtypeStruct((B, D), jnp.float32),
    mesh=_mesh,
    scratch_types=[
        pltpu.VMEM((NCHUNK, CHUNK), jnp.int32),
        pltpu.VMEM((NBUF, CHUNK, D), jnp.float32),
        pltpu.SemaphoreType.DMA((NBUF,)),
        pltpu.SemaphoreType.DMA((NBUF,)),
    ],
    compiler_params=pltpu.CompilerParams(use_tc_tiling_on_sc=False),
)
def _gather(idx_hbm, table_hbm, out_hbm, idx_v, rows_v, gsem, osem):
    wid = lax.axis_index("s") * NC + lax.axis_index("c")
    pltpu.sync_copy(idx_hbm.at[wid], idx_v)
    base = wid * BPW

    def gather_start(jj, b):
        pltpu.async_copy(table_hbm.at[idx_v.at[jj]], rows_v.at[b], gsem.at[b])

    def gather_wait(jj, b):
        pltpu.make_async_copy(
            table_hbm.at[idx_v.at[jj]], rows_v.at[b], gsem.at[b]
        ).wait()

    def out_start(jj, b):
        pltpu.async_copy(
            rows_v.at[b], out_hbm.at[pl.ds(base + jj * CHUNK, CHUNK)], osem.at[b]
        )

    def out_wait(jj, b):
        pltpu.make_async_copy(
            rows_v.at[b], out_hbm.at[pl.ds(base + jj * CHUNK, CHUNK)], osem.at[b]
        ).wait()

    for b in range(NBUF - 1):
        gather_start(b, b)

    @pl.loop(0, NCHUNK, step=NBUF)
    def _group(j):
        for b in range(NBUF):
            jj = j + b
            bn = (b + NBUF - 1) % NBUF
            jn = jj + NBUF - 1

            @pl.when(jn < NCHUNK)
            def _():
                @pl.when(jn >= NBUF)
                def _():
                    out_wait(jn - NBUF, bn)
                gather_start(jn, bn)

            gather_wait(jj, b)
            out_start(jj, b)

    for b in range(NBUF):
        out_wait(NCHUNK - NBUF + b, (NCHUNK - NBUF + b) % NBUF)


def kernel(x, embed_table):
    idx = x.T.astype(jnp.int32).reshape(NW, NCHUNK, CHUNK)
    out = _gather(idx, embed_table)
    s, b = x.shape[1], x.shape[0]
    return out.reshape(s, b, D).transpose(1, 0, 2)


# speedup vs baseline: 1.9466x; 1.0008x over previous
import functools

import jax
import jax.numpy as jnp
from jax import lax
from jax.experimental import pallas as pl
from jax.experimental.pallas import tpu as pltpu
from jax.experimental.pallas import tpu_sc as plsc

NUM_EMB = 1000000
D = 32
NC, NS = 2, 16
NW = NC * NS
B = 16384 * 50
BPW = B // NW
CHUNK = 512
NCHUNK = BPW // CHUNK
NBUF = 5

_mesh = plsc.VectorSubcoreMesh(
    core_axis_name="c", subcore_axis_name="s", num_cores=NC, num_subcores=NS
)


@functools.partial(
    pl.kernel,
    out_type=jax.ShapeDtypeStruct((B, D), jnp.float32),
    mesh=_mesh,
    scratch_types=[
        pltpu.VMEM((NCHUNK, CHUNK), jnp.int32),
        pltpu.VMEM((NBUF, CHUNK, D), jnp.float32),
        pltpu.SemaphoreType.DMA((NBUF,)),
        pltpu.SemaphoreType.DMA((NBUF,)),
    ],
    compiler_params=pltpu.CompilerParams(use_tc_tiling_on_sc=False),
)
def _gather(idx_hbm, table_hbm, out_hbm, idx_v, rows_v, gsem, osem):
    wid = lax.axis_index("s") * NC + lax.axis_index("c")
    pltpu.sync_copy(idx_hbm.at[:, wid], idx_v)

    def gather_start(jj, b):
        pltpu.async_copy(table_hbm.at[idx_v.at[jj]], rows_v.at[b], gsem.at[b])

    def gather_wait(jj, b):
        pltpu.make_async_copy(
            table_hbm.at[idx_v.at[jj]], rows_v.at[b], gsem.at[b]
        ).wait()

    def out_start(jj, b):
        pltpu.async_copy(
            rows_v.at[b],
            out_hbm.at[pl.ds(jj * NW * CHUNK + wid * CHUNK, CHUNK)],
            osem.at[b],
        )

    def out_wait(jj, b):
        pltpu.make_async_copy(
            rows_v.at[b],
            out_hbm.at[pl.ds(jj * NW * CHUNK + wid * CHUNK, CHUNK)],
            osem.at[b],
        ).wait()

    for b in range(NBUF - 1):
        gather_start(b, b)

    @pl.loop(0, NCHUNK, step=NBUF)
    def _group(j):
        for b in range(NBUF):
            jj = j + b
            bn = (b + NBUF - 1) % NBUF
            jn = jj + NBUF - 1

            @pl.when(jn < NCHUNK)
            def _():
                @pl.when(jn >= NBUF)
                def _():
                    out_wait(jn - NBUF, bn)
                gather_start(jn, bn)

            gather_wait(jj, b)
            out_start(jj, b)

    for b in range(NBUF):
        out_wait(NCHUNK - NBUF + b, (NCHUNK - NBUF + b) % NBUF)


def kernel(x, embed_table):
    idx = x.T.astype(jnp.int32).reshape(NCHUNK, NW, CHUNK)
    out = _gather(idx, embed_table)
    s, b = x.shape[1], x.shape[0]
    return out.reshape(s, b, D).transpose(1, 0, 2)
